# Initial kernel scaffold; baseline (speedup 1.0000x reference)
import jax
import jax.numpy as jnp
from jax.experimental import pallas as pl


def kernel(superPoint_feat, rawPoint_feat, raw_to_super_index, label_inds):
    raise NotImplementedError("write your pallas kernel here")


# speedup vs baseline: 6.9933x; 6.9933x over previous
import functools

import jax
import jax.numpy as jnp
from jax.experimental import pallas as pl
from jax.experimental.pallas import tpu as pltpu

_DELTA_DIST = 1.2
_VAR_W, _DIST_W, _REG_W, _ENT_W, _LOSS_W = 1.0, 1.0, 0.001, 1.0, 0.1
_NUM_CLASSES = 20


def _safe_sqrt(sq):
    pos = sq > 0
    r = jnp.sqrt(jnp.where(pos, sq, 1.0))
    return jnp.where(pos, r, 0.0)


def _loss_kernel(sp_ref, raw_ref, idx_ref, lab_ref, out_ref, seg_acc, hist_acc,
                 *, num_blocks, block_b, num_sp):
    j = pl.program_id(0)

    @pl.when(j == 0)
    def _init():
        seg_acc[...] = jnp.zeros_like(seg_acc)
        hist_acc[...] = jnp.zeros_like(hist_acc)

    sp = sp_ref[...]
    n2 = jnp.sum(sp * sp, axis=1, keepdims=True)
    spn = sp / jnp.maximum(jnp.sqrt(n2), 1e-12)

    raw = raw_ref[...]
    rn2 = jnp.sum(raw * raw, axis=1, keepdims=True)
    rawn = raw / jnp.maximum(jnp.sqrt(rn2), 1e-12)

    idx = idx_ref[0, 0, :]
    onehot = (idx[:, None] == jax.lax.broadcasted_iota(
        jnp.int32, (1, num_sp), 1)).astype(jnp.bfloat16)

    seln = jnp.dot(onehot, spn.astype(jnp.bfloat16),
                   preferred_element_type=jnp.float32)
    d = rawn - seln
    dsq = jnp.sum(d * d, axis=1, keepdims=True)
    dist = _safe_sqrt(dsq)

    seg_part = jax.lax.dot_general(
        dist.astype(jnp.bfloat16), onehot,
        (((0,), (0,)), ((), ())), preferred_element_type=jnp.float32)
    seg_acc[...] += seg_part

    lab = lab_ref[0, 0, :]
    lab_oh_t = (jax.lax.broadcasted_iota(jnp.int32, (_NUM_CLASSES, 1), 0)
                == lab[None, :]).astype(jnp.bfloat16)
    hist_part = jax.lax.dot_general(
        lab_oh_t, onehot,
        (((1,), (0,)), ((), ())), preferred_element_type=jnp.float32)
    hist_acc[...] += hist_part

    @pl.when(j == num_blocks - 1)
    def _finish():
        hist = hist_acc[...]
        counts = jnp.sum(hist, axis=0, keepdims=True)
        seg = seg_acc[...]

        per_var = seg / jnp.maximum(counts, 1.0)
        l_var = jnp.sum(jnp.maximum(per_var, 0.0)) / num_sp

        label_sums = counts + 1e-8
        probs = hist / label_sums
        entropy = -jnp.sum(probs * jnp.log(probs + 1e-8), axis=0,
                           keepdims=True)
        valid = (label_sums > 0).astype(jnp.float32)
        l_entropy = jnp.sum(entropy * valid) / jnp.maximum(jnp.sum(valid), 1.0)

        ci = jax.lax.broadcasted_iota(jnp.int32, (_NUM_CLASSES, num_sp), 0)
        mx = jnp.max(hist, axis=0, keepdims=True)
        sel_lab = jnp.min(jnp.where(hist == mx, ci, _NUM_CLASSES), axis=0,
                          keepdims=True)
        lab_mat = (ci == jnp.broadcast_to(sel_lab, (_NUM_CLASSES, num_sp))
                   ).astype(jnp.float32)
        same_label = jax.lax.dot_general(
            lab_mat, lab_mat, (((0,), (0,)), ((), ())),
            preferred_element_type=jnp.float32)

        pair_valid = jax.lax.dot_general(
            valid, valid, (((0,), (0,)), ((), ())),
            preferred_element_type=jnp.float32)

        ri = jax.lax.broadcasted_iota(jnp.int32, (num_sp, num_sp), 0)
        cj = jax.lax.broadcasted_iota(jnp.int32, (num_sp, num_sp), 1)
        off_diag = (ri != cj).astype(jnp.float32)

        mask = (1.0 - same_label) * off_diag * pair_valid

        gram = jax.lax.dot_general(
            spn, spn, (((1,), (1,)), ((), ())),
            preferred_element_type=jnp.float32)
        gd = jnp.sum(spn * spn, axis=1, keepdims=True)
        cdsq = gd + jnp.transpose(gd) - 2.0 * gram
        center_dist = _safe_sqrt(cdsq)
        vals = jnp.maximum(_DELTA_DIST - center_dist, 0.0) ** 2
        l_dist = jnp.sum(vals * mask) / jnp.maximum(jnp.sum(mask), 1.0)

        l_reg = jnp.sum(_safe_sqrt(n2)) / num_sp

        total = (_VAR_W * l_var + _DIST_W * l_dist + _REG_W * l_reg
                 + _ENT_W * l_entropy)
        out_ref[...] = jnp.reshape(total * _LOSS_W, (1, 1))


def kernel(superPoint_feat, rawPoint_feat, raw_to_super_index, label_inds):
    num_sp, feat = superPoint_feat.shape
    n = rawPoint_feat.shape[0]

    block_b = n
    for cand in (2000, 2048, 1024, 1000, 512, 200, 100):
        if n % cand == 0:
            block_b = cand
            break
    num_blocks = n // block_b

    idx3 = raw_to_super_index.reshape(num_blocks, 1, block_b)
    lab3 = label_inds.reshape(num_blocks, 1, block_b)

    body = functools.partial(_loss_kernel, num_blocks=num_blocks,
                             block_b=block_b, num_sp=num_sp)

    out = pl.pallas_call(
        body,
        grid=(num_blocks,),
        in_specs=[
            pl.BlockSpec((num_sp, feat), lambda j: (0, 0)),
            pl.BlockSpec((block_b, feat), lambda j: (j, 0)),
            pl.BlockSpec((1, 1, block_b), lambda j: (j, 0, 0)),
            pl.BlockSpec((1, 1, block_b), lambda j: (j, 0, 0)),
        ],
        out_specs=pl.BlockSpec((1, 1), lambda j: (0, 0)),
        out_shape=jax.ShapeDtypeStruct((1, 1), jnp.float32),
        scratch_shapes=[
            pltpu.VMEM((1, num_sp), jnp.float32),
            pltpu.VMEM((_NUM_CLASSES, num_sp), jnp.float32),
        ],
        compiler_params=pltpu.CompilerParams(
            dimension_semantics=("arbitrary",)),
    )(superPoint_feat, rawPoint_feat, idx3, lab3)
    return out[0, 0]
